# baseline (speedup 1.0000x reference)
import jax
import jax.numpy as jnp
from jax import lax
from jax.experimental import pallas as pl
from jax.experimental.pallas import tpu as pltpu
from jax.experimental.pallas import tpu_sc as plsc

_DIM = 64
_NC = 2
_NS = 16
_NW = _NC * _NS
_LANES = 16
_NBUF = 4
_SUPER = 8


def _body(B, L, tid_hbm, cid_hbm, ww_hbm, wc_hbm, out_hbm,
          tidx_v, cidx_v, outbuf_v, *rest):
    trows = rest[:_NBUF]
    crows = rest[_NBUF:2 * _NBUF]
    sem_t = rest[2 * _NBUF:3 * _NBUF]
    sem_c = rest[3 * _NBUF:4 * _NBUF]
    sem_it, sem_ic = rest[4 * _NBUF:4 * _NBUF + 2]

    rows_pt = B // _NW
    per_w = rows_pt * L
    groups = (L + _LANES - 1) // _LANES
    padrows = groups * _LANES
    supers = rows_pt // _SUPER

    wid = lax.axis_index("s") * _NC + lax.axis_index("c")
    row0 = wid * rows_pt

    def issue(chunk, slot):
        irow = lax.rem(chunk, 2 * _SUPER)
        ct = pltpu.make_async_copy(ww_hbm.at[tidx_v.at[irow]],
                                   trows[slot].at[pl.ds(0, L), :], sem_t[slot])
        cc = pltpu.make_async_copy(wc_hbm.at[cidx_v.at[irow]],
                                   crows[slot].at[pl.ds(0, L), :], sem_c[slot])
        ct.start()
        cc.start()

    def issue_idx(sup, sync):
        half = lax.rem(sup, 2) * _SUPER
        src_t = tid_hbm.at[pl.ds(row0 + sup * _SUPER, _SUPER), :]
        src_c = cid_hbm.at[pl.ds(row0 + sup * _SUPER, _SUPER), :]
        dst_t = tidx_v.at[pl.ds(half, _SUPER), :]
        dst_c = cidx_v.at[pl.ds(half, _SUPER), :]
        if sync:
            pltpu.sync_copy(src_t, dst_t)
            pltpu.sync_copy(src_c, dst_c)
        else:
            pltpu.make_async_copy(src_t, dst_t, sem_it).start()
            pltpu.make_async_copy(src_c, dst_c, sem_ic).start()

    def wait_idx():
        pltpu.make_async_copy(tid_hbm.at[pl.ds(0, _SUPER), :],
                              tidx_v.at[pl.ds(0, _SUPER), :], sem_it).wait()
        pltpu.make_async_copy(cid_hbm.at[pl.ds(0, _SUPER), :],
                              cidx_v.at[pl.ds(0, _SUPER), :], sem_ic).wait()

    issue_idx(0, True)
    for b in range(_NBUF - 1):
        issue(b, b)

    def super_body(s, carry):
        @pl.when(s + 1 < supers)
        def _stage():
            issue_idx(s + 1, False)

        for b in range(_SUPER):
            i = s * _SUPER + b
            j = i + _NBUF - 1
            slot = (b + _NBUF - 1) % _NBUF

            if b == _SUPER - (_NBUF - 1):
                @pl.when(s + 1 < supers)
                def _wait_stage():
                    wait_idx()

            @pl.when(j < supers * _SUPER)
            def _issue():
                issue(j, slot)

            cslot = b % _NBUF
            pltpu.make_async_copy(ww_hbm.at[pl.ds(0, L), :],
                                  trows[cslot].at[pl.ds(0, L), :],
                                  sem_t[cslot]).wait()
            pltpu.make_async_copy(wc_hbm.at[pl.ds(0, L), :],
                                  crows[cslot].at[pl.ds(0, L), :],
                                  sem_c[cslot]).wait()

            def group_body(g, c2, _slot=cslot, _i=i):
                lane = lax.iota(jnp.int32, _LANES)
                rows = g * _LANES + lane
                acc = jnp.zeros((_LANES,), jnp.float32)
                for d in range(_DIM):
                    cols = (lane + d) & (_DIM - 1)
                    tv = plsc.load_gather(trows[_slot], [rows, cols])
                    cv = plsc.load_gather(crows[_slot], [rows, cols])
                    acc = acc + tv * cv
                score = 1.0 / (1.0 + jnp.exp(-acc))
                outbuf_v[pl.ds(_i * L + g * _LANES, _LANES)] = score
                return c2

            lax.fori_loop(0, groups, group_body, 0)
        return carry

    lax.fori_loop(0, supers, super_body, 0)
    pltpu.sync_copy(outbuf_v.at[pl.ds(0, per_w)],
                    out_hbm.at[pl.ds(wid * per_w, per_w)])


def kernel(target_word_ids, context_word_ids, W_words, W_context):
    B, L = target_word_ids.shape
    total = B * L
    rows_pt = B // _NW
    assert B % (_NW * _SUPER) == 0 and (rows_pt // _SUPER) >= 2
    per_w = rows_pt * L
    groups = (L + _LANES - 1) // _LANES
    padrows = groups * _LANES

    mesh = plsc.VectorSubcoreMesh(core_axis_name="c", subcore_axis_name="s")
    row_bufs = [pltpu.VMEM((padrows, _DIM), jnp.float32)
                for _ in range(2 * _NBUF)]
    sems = [pltpu.SemaphoreType.DMA for _ in range(2 * _NBUF + 2)]
    k = pl.kernel(
        lambda *args: _body(B, L, *args),
        out_type=jax.ShapeDtypeStruct((total,), jnp.float32),
        mesh=mesh,
        compiler_params=pltpu.CompilerParams(
            needs_layout_passes=False, use_tc_tiling_on_sc=False),
        scratch_types=[
            pltpu.VMEM((2 * _SUPER, L), jnp.int32),
            pltpu.VMEM((2 * _SUPER, L), jnp.int32),
            pltpu.VMEM((per_w + _LANES, ), jnp.float32),
        ] + row_bufs + sems,
    )
    out = k(target_word_ids.astype(jnp.int32), context_word_ids.astype(jnp.int32),
            W_words.astype(jnp.float32), W_context.astype(jnp.float32))
    return out.reshape(B, L)


# speedup vs baseline: 1.1089x; 1.1089x over previous
import functools

import jax
import jax.numpy as jnp
from jax import lax
from jax.experimental import pallas as pl
from jax.experimental.pallas import tpu as pltpu
from jax.experimental.pallas import tpu_sc as plsc

_DIM = 64
_NC = 2
_NS = 16
_NW = _NC * _NS
_LANES = 16
_CHUNK = 64
_GROUPS = _CHUNK // _LANES
_NBUF = 4


def _body(total, args):
    (tid_hbm, cid_hbm, ww_hbm, wc_hbm, out_hbm,
     tids_v, cids_v, outbuf_v) = args[:8]
    trows = args[8:8 + _NBUF]
    crows = args[8 + _NBUF:8 + 2 * _NBUF]
    sem_t = args[8 + 2 * _NBUF:8 + 3 * _NBUF]
    sem_c = args[8 + 3 * _NBUF:8 + 4 * _NBUF]

    wid = lax.axis_index("s") * _NC + lax.axis_index("c")
    per_w = total // _NW
    nchunk = per_w // _CHUNK
    supers = nchunk // _NBUF
    base_w = wid * per_w

    pltpu.sync_copy(tid_hbm.at[pl.ds(base_w, per_w)], tids_v)
    pltpu.sync_copy(cid_hbm.at[pl.ds(base_w, per_w)], cids_v)

    def issue(chunk, b):
        idx_t = tids_v.at[pl.ds(chunk * _CHUNK, _CHUNK)]
        idx_c = cids_v.at[pl.ds(chunk * _CHUNK, _CHUNK)]
        ct = pltpu.make_async_copy(ww_hbm.at[idx_t], trows[b], sem_t[b])
        cc = pltpu.make_async_copy(wc_hbm.at[idx_c], crows[b], sem_c[b])
        ct.start()
        cc.start()

    for b in range(_NBUF - 1):
        issue(b, b)

    def super_body(p, carry):
        for b in range(_NBUF):
            i = p * _NBUF + b
            j = i + _NBUF - 1

            @pl.when(j < nchunk)
            def _issue():
                issue(j, (b + _NBUF - 1) % _NBUF)

            pltpu.make_async_copy(ww_hbm.at[pl.ds(0, _CHUNK), :],
                                  trows[b], sem_t[b]).wait()
            pltpu.make_async_copy(wc_hbm.at[pl.ds(0, _CHUNK), :],
                                  crows[b], sem_c[b]).wait()

            def group_body(g, c2, _b=b, _i=i):
                lane = lax.iota(jnp.int32, _LANES)
                rows = g * _LANES + lane
                acc = jnp.zeros((_LANES,), jnp.float32)
                for d in range(_DIM):
                    cols = (lane + d) & (_DIM - 1)
                    tv = plsc.load_gather(trows[_b], [rows, cols])
                    cv = plsc.load_gather(crows[_b], [rows, cols])
                    acc = acc + tv * cv
                score = 1.0 / (1.0 + jnp.exp(-acc))
                outbuf_v[pl.ds(_i * _CHUNK + g * _LANES, _LANES)] = score
                return c2

            lax.fori_loop(0, _GROUPS, group_body, 0)
        return carry

    lax.fori_loop(0, supers, super_body, 0)
    pltpu.sync_copy(outbuf_v, out_hbm.at[pl.ds(base_w, per_w)])


def kernel(target_word_ids, context_word_ids, W_words, W_context):
    B, L = target_word_ids.shape
    total = B * L
    per_w = total // _NW
    assert total % (_NW * _CHUNK * _NBUF) == 0
    tid = target_word_ids.reshape(total).astype(jnp.int32)
    cid = context_word_ids.reshape(total).astype(jnp.int32)

    mesh = plsc.VectorSubcoreMesh(core_axis_name="c", subcore_axis_name="s")
    row_bufs = [pltpu.VMEM((_CHUNK, _DIM), jnp.float32)
                for _ in range(2 * _NBUF)]
    sems = [pltpu.SemaphoreType.DMA for _ in range(2 * _NBUF)]
    k = pl.kernel(
        lambda *args: _body(total, args),
        out_type=jax.ShapeDtypeStruct((total,), jnp.float32),
        mesh=mesh,
        compiler_params=pltpu.CompilerParams(
            needs_layout_passes=False, use_tc_tiling_on_sc=False),
        scratch_types=[
            pltpu.VMEM((per_w,), jnp.int32),
            pltpu.VMEM((per_w,), jnp.int32),
            pltpu.VMEM((per_w,), jnp.float32),
        ] + row_bufs + sems,
    )
    out = k(tid, cid, W_words.astype(jnp.float32), W_context.astype(jnp.float32))
    return out.reshape(B, L)
